# baseline (speedup 1.0000x reference)
import functools

import jax
import jax.numpy as jnp
from jax import lax
from jax.experimental import pallas as pl
from jax.experimental.pallas import tpu as pltpu
from jax.experimental.pallas import tpu_sc as plsc

D = 16
NC = 2
NS = 16
NW = NC * NS
CHUNK = 128


def _sc_gather_call(uid2d, iid2d, t_ug, t_ig, t_um, t_im):
    nb = uid2d.shape[0]
    ch = nb // NW
    out_t = [jax.ShapeDtypeStruct((nb, CHUNK, D), jnp.float32)] * 4
    mesh = plsc.VectorSubcoreMesh(core_axis_name="c", subcore_axis_name="s")

    @functools.partial(
        pl.kernel,
        out_type=out_t,
        mesh=mesh,
        compiler_params=pltpu.CompilerParams(use_tc_tiling_on_sc=False),
        scratch_types=[
            pltpu.VMEM((ch, CHUNK), jnp.int32),
            pltpu.VMEM((ch, CHUNK), jnp.int32),
            pltpu.VMEM((ch, CHUNK, D), jnp.float32),
            pltpu.VMEM((ch, CHUNK, D), jnp.float32),
            pltpu.VMEM((ch, CHUNK, D), jnp.float32),
            pltpu.VMEM((ch, CHUNK, D), jnp.float32),
            pltpu.SemaphoreType.DMA,
        ],
    )
    def body(uid_h, iid_h, ug_h, ig_h, um_h, im_h,
             o_ug, o_ig, o_um, o_im,
             idx_u, idx_i, b_ug, b_ig, b_um, b_im, sem):
        wid = lax.axis_index("s") * NC + lax.axis_index("c")
        base = wid * ch
        pltpu.sync_copy(uid_h.at[pl.ds(base, ch)], idx_u)
        pltpu.sync_copy(iid_h.at[pl.ds(base, ch)], idx_i)
        copies = []
        for j in range(ch):
            copies.append(pltpu.async_copy(ug_h.at[idx_u.at[j]], b_ug.at[j], sem))
            copies.append(pltpu.async_copy(ig_h.at[idx_i.at[j]], b_ig.at[j], sem))
            copies.append(pltpu.async_copy(um_h.at[idx_u.at[j]], b_um.at[j], sem))
            copies.append(pltpu.async_copy(im_h.at[idx_i.at[j]], b_im.at[j], sem))
        for c in copies:
            c.wait()
        pltpu.sync_copy(b_ug, o_ug.at[pl.ds(base, ch)])
        pltpu.sync_copy(b_ig, o_ig.at[pl.ds(base, ch)])
        pltpu.sync_copy(b_um, o_um.at[pl.ds(base, ch)])
        pltpu.sync_copy(b_im, o_im.at[pl.ds(base, ch)])

    return body(uid2d, iid2d, t_ug, t_ig, t_um, t_im)


def _tc_dense_body(gu, gi, um, im, w1, b1, w2, b2, wl, bl, out):
    x_gmf = gu[...] * gi[...]
    h1 = jnp.dot(um[...], w1[:D], preferred_element_type=jnp.float32)
    h1 += jnp.dot(im[...], w1[D:], preferred_element_type=jnp.float32)
    h1 = jnp.maximum(h1 + b1[...], 0.0)
    h2 = jnp.dot(h1, w2[...], preferred_element_type=jnp.float32) + b2[...]
    h2 = jnp.maximum(h2, 0.0)
    o = jnp.dot(x_gmf, wl[:D], preferred_element_type=jnp.float32)
    o += jnp.dot(h2, wl[D:], preferred_element_type=jnp.float32)
    out[...] = o + bl[...]


def _tc_dense_call(gu, gi, um, im, W1, b1, W2, b2, WL, bL):
    B = gu.shape[0]
    BR = 2048
    grid = (B // BR,)
    row_spec = pl.BlockSpec((BR, D), lambda i: (i, 0))
    full = lambda a: pl.BlockSpec(a.shape, lambda i: tuple(0 for _ in a.shape))
    return pl.pallas_call(
        _tc_dense_body,
        grid=grid,
        in_specs=[row_spec, row_spec, row_spec, row_spec,
                  full(W1), full(b1), full(W2), full(b2), full(WL), full(bL)],
        out_specs=pl.BlockSpec((BR, 1), lambda i: (i, 0)),
        out_shape=jax.ShapeDtypeStruct((B, 1), jnp.float32),
    )(gu, gi, um, im, W1, b1, W2, b2, WL, bL)


def kernel(X, user_gmf, item_gmf, user_mlp, item_mlp, W1, b1, W2, b2, WL, bL):
    B = X.shape[0]
    nb = B // CHUNK
    uid2d = X[:, 0].astype(jnp.int32).reshape(nb, CHUNK)
    iid2d = X[:, 1].astype(jnp.int32).reshape(nb, CHUNK)
    ug, ig, um, im = _sc_gather_call(uid2d, iid2d,
                                     user_gmf, item_gmf, user_mlp, item_mlp)
    out = _tc_dense_call(
        ug.reshape(B, D), ig.reshape(B, D), um.reshape(B, D), im.reshape(B, D),
        W1, b1.reshape(1, D), W2, b2.reshape(1, D // 2),
        WL, bL.reshape(1, 1))
    return out


# speedup vs baseline: 1.4104x; 1.4104x over previous
import functools

import jax
import jax.numpy as jnp
from jax import lax
from jax.experimental import pallas as pl
from jax.experimental.pallas import tpu as pltpu
from jax.experimental.pallas import tpu_sc as plsc

D = 16
NC = 2
NS = 16
NW = NC * NS
CHUNK = 128
L = 16


def _sc_gather_call(uid, iid, t_ug, t_ig, t_um, t_im):
    B = uid.shape[0]
    ch = B // (NW * CHUNK)
    out_t = [jax.ShapeDtypeStruct((B, D), jnp.float32)] * 4
    mesh = plsc.VectorSubcoreMesh(core_axis_name="c", subcore_axis_name="s")

    @functools.partial(
        pl.kernel,
        out_type=out_t,
        mesh=mesh,
        scratch_types=[
            pltpu.VMEM((CHUNK,), jnp.int32),
            pltpu.VMEM((CHUNK,), jnp.int32),
            pltpu.VMEM((CHUNK, D), jnp.float32),
            pltpu.VMEM((CHUNK, D), jnp.float32),
            pltpu.VMEM((CHUNK, D), jnp.float32),
            pltpu.VMEM((CHUNK, D), jnp.float32),
            pltpu.SemaphoreType.DMA,
        ],
    )
    def body(uid_h, iid_h, ug_h, ig_h, um_h, im_h,
             o_ug, o_ig, o_um, o_im,
             idx_u, idx_i, b_ug, b_ig, b_um, b_im, sem):
        wid = lax.axis_index("s") * NC + lax.axis_index("c")
        for j in range(ch):
            base = (wid * ch + j) * CHUNK
            pltpu.sync_copy(uid_h.at[pl.ds(base, CHUNK)], idx_u)
            pltpu.sync_copy(iid_h.at[pl.ds(base, CHUNK)], idx_i)

            def fire(q, _):
                uv = idx_u[pl.ds(q * L, L)]
                iv = idx_i[pl.ds(q * L, L)]
                for k in range(L):
                    r = q * L + k
                    pltpu.async_copy(ug_h.at[pl.ds(uv[k], 1)], b_ug.at[pl.ds(r, 1)], sem)
                    pltpu.async_copy(ig_h.at[pl.ds(iv[k], 1)], b_ig.at[pl.ds(r, 1)], sem)
                    pltpu.async_copy(um_h.at[pl.ds(uv[k], 1)], b_um.at[pl.ds(r, 1)], sem)
                    pltpu.async_copy(im_h.at[pl.ds(iv[k], 1)], b_im.at[pl.ds(r, 1)], sem)
                return _

            lax.fori_loop(0, CHUNK // L, fire, 0)

            def drain(r, _):
                for buf in (b_ug, b_ig, b_um, b_im):
                    pltpu.make_async_copy(
                        ug_h.at[pl.ds(0, 1)], buf.at[pl.ds(r, 1)], sem).wait()
                return _

            lax.fori_loop(0, CHUNK, drain, 0)
            pltpu.sync_copy(b_ug, o_ug.at[pl.ds(base, CHUNK)])
            pltpu.sync_copy(b_ig, o_ig.at[pl.ds(base, CHUNK)])
            pltpu.sync_copy(b_um, o_um.at[pl.ds(base, CHUNK)])
            pltpu.sync_copy(b_im, o_im.at[pl.ds(base, CHUNK)])

    return body(uid, iid, t_ug, t_ig, t_um, t_im)


def _tc_dense_body(gu, gi, um, im, w1, b1, w2, b2, wl, bl, out):
    x_gmf = gu[...] * gi[...]
    h1 = jnp.dot(um[...], w1[:D], preferred_element_type=jnp.float32)
    h1 += jnp.dot(im[...], w1[D:], preferred_element_type=jnp.float32)
    h1 = jnp.maximum(h1 + b1[...], 0.0)
    h2 = jnp.dot(h1, w2[...], preferred_element_type=jnp.float32) + b2[...]
    h2 = jnp.maximum(h2, 0.0)
    o = jnp.dot(x_gmf, wl[:D], preferred_element_type=jnp.float32)
    o += jnp.dot(h2, wl[D:], preferred_element_type=jnp.float32)
    out[...] = o + bl[...]


def _tc_dense_call(gu, gi, um, im, W1, b1, W2, b2, WL, bL):
    B = gu.shape[0]
    BR = 2048
    grid = (B // BR,)
    row_spec = pl.BlockSpec((BR, D), lambda i: (i, 0))
    full = lambda a: pl.BlockSpec(a.shape, lambda i: tuple(0 for _ in a.shape))
    return pl.pallas_call(
        _tc_dense_body,
        grid=grid,
        in_specs=[row_spec, row_spec, row_spec, row_spec,
                  full(W1), full(b1), full(W2), full(b2), full(WL), full(bL)],
        out_specs=pl.BlockSpec((BR, 1), lambda i: (i, 0)),
        out_shape=jax.ShapeDtypeStruct((B, 1), jnp.float32),
    )(gu, gi, um, im, W1, b1, W2, b2, WL, bL)


def kernel(X, user_gmf, item_gmf, user_mlp, item_mlp, W1, b1, W2, b2, WL, bL):
    X = X.astype(jnp.int32)
    ug, ig, um, im = _sc_gather_call(X[:, 0], X[:, 1],
                                     user_gmf, item_gmf, user_mlp, item_mlp)
    out = _tc_dense_call(
        ug, ig, um, im,
        W1, b1.reshape(1, D), W2, b2.reshape(1, D // 2),
        WL, bL.reshape(1, 1))
    return out
